# baseline (speedup 1.0000x reference)
import functools

import jax
import jax.numpy as jnp
from jax import lax
from jax.experimental import pallas as pl
from jax.experimental.pallas import tpu as pltpu
from jax.experimental.pallas import tpu_sc as plsc

N = 800000
NSTRUCT = 4000
NSEG = 4096
NC = 2
NS = 16
NW = NC * NS
BLK = 640
NBLK = N // BLK
D1 = 24
D2 = 40
GRP = BLK // 16

_mesh = plsc.VectorSubcoreMesh(core_axis_name="c", subcore_axis_name="s")


@functools.partial(
    pl.kernel,
    out_type=jax.ShapeDtypeStruct((NW, 3 * NSEG), jnp.float32),
    mesh=_mesh,
    scratch_types=[
        pltpu.VMEM((BLK * D1,), jnp.float32),
        pltpu.VMEM((BLK * D2,), jnp.float32),
        pltpu.VMEM((BLK,), jnp.int32),
        pltpu.VMEM((3 * NSEG,), jnp.float32),
    ],
    compiler_params=pltpu.CompilerParams(needs_layout_passes=False),
)
def _partial_kernel(l1_hbm, l2_hbm, ids_hbm, part_hbm, l1b, l2b, idsb, acc):
    wid = lax.axis_index("s") * NC + lax.axis_index("c")

    zeros16 = jnp.zeros((16,), jnp.float32)

    def zero_body(i, _):
        acc[pl.ds(i * 16, 16)] = zeros16
        return _

    lax.fori_loop(0, 3 * NSEG // 16, zero_body, None)

    lane = lax.iota(jnp.int32, 16)
    idx1_base = lane * D1
    idx2_base = lane * D2
    ones16 = jnp.ones((16,), jnp.float32)

    def group_body(g, _):
        b1 = g * (16 * D1)
        b2 = g * (16 * D2)
        acc1 = zeros16
        acc2 = zeros16
        for j in range(D1):
            v = plsc.load_gather(l1b, [idx1_base + (b1 + j)])
            acc1 = acc1 + v * v
        for j in range(D2):
            v = plsc.load_gather(l2b, [idx2_base + (b2 + j)])
            acc2 = acc2 + v * v
        ids_v = idsb[pl.ds(g * 16, 16)]
        plsc.addupdate_scatter(acc, [ids_v], acc1)
        plsc.addupdate_scatter(acc, [ids_v + NSEG], acc2)
        plsc.addupdate_scatter(acc, [ids_v + 2 * NSEG], ones16)
        return _

    def block_body(i, _):
        blk = wid + i * NW
        pltpu.sync_copy(l1_hbm.at[pl.ds(blk * (BLK * D1), BLK * D1)], l1b)
        pltpu.sync_copy(l2_hbm.at[pl.ds(blk * (BLK * D2), BLK * D2)], l2b)
        pltpu.sync_copy(ids_hbm.at[pl.ds(blk * BLK, BLK)], idsb)
        lax.fori_loop(0, GRP, group_body, None)
        return _

    nblk_w = (NBLK + NW - 1 - wid) // NW
    lax.fori_loop(0, nblk_w, block_body, None)

    pltpu.sync_copy(acc, part_hbm.at[wid])


_CHUNK = NSEG // NW


@functools.partial(
    pl.kernel,
    out_type=jax.ShapeDtypeStruct((2 * NSEG,), jnp.float32),
    mesh=_mesh,
    scratch_types=[
        pltpu.VMEM((_CHUNK,), jnp.float32),
        pltpu.VMEM((_CHUNK,), jnp.float32),
        pltpu.VMEM((_CHUNK,), jnp.float32),
        pltpu.VMEM((_CHUNK,), jnp.float32),
    ],
    compiler_params=pltpu.CompilerParams(needs_layout_passes=False),
)
def _finalize_kernel(part_hbm, out_hbm, s1, s2, cnt, tmp):
    wid = lax.axis_index("s") * NC + lax.axis_index("c")
    base = wid * _CHUNK
    zeros16 = jnp.zeros((16,), jnp.float32)

    def zero_body(i, _):
        s1[pl.ds(i * 16, 16)] = zeros16
        s2[pl.ds(i * 16, 16)] = zeros16
        cnt[pl.ds(i * 16, 16)] = zeros16
        return _

    lax.fori_loop(0, _CHUNK // 16, zero_body, None)

    def add_into(dst, i):
        def body(k, _):
            sl = pl.ds(k * 16, 16)
            dst[sl] = dst[sl] + tmp[sl]
            return _

        lax.fori_loop(0, _CHUNK // 16, body, None)

    def red_body(i, _):
        off = i * (3 * NSEG) + base
        pltpu.sync_copy(part_hbm.at[pl.ds(off, _CHUNK)], tmp)
        add_into(s1, i)
        pltpu.sync_copy(part_hbm.at[pl.ds(off + NSEG, _CHUNK)], tmp)
        add_into(s2, i)
        pltpu.sync_copy(part_hbm.at[pl.ds(off + 2 * NSEG, _CHUNK)], tmp)
        add_into(cnt, i)
        return _

    lax.fori_loop(0, NW, red_body, None)

    ones16 = jnp.ones((16,), jnp.float32)

    def div_body(k, _):
        sl = pl.ds(k * 16, 16)
        c = jnp.maximum(cnt[sl], ones16)
        s1[sl] = s1[sl] / c
        s2[sl] = s2[sl] / c
        return _

    lax.fori_loop(0, _CHUNK // 16, div_body, None)

    pltpu.sync_copy(s1, out_hbm.at[pl.ds(base, _CHUNK)])
    pltpu.sync_copy(s2, out_hbm.at[pl.ds(NSEG + base, _CHUNK)])


def kernel(spex_l1, spex_l2, structure_ids):
    l1_flat = spex_l1.reshape(N * D1)
    l2_flat = spex_l2.reshape(N * D2)
    ids = structure_ids.astype(jnp.int32)
    part = _partial_kernel(l1_flat, l2_flat, ids)
    out = _finalize_kernel(part.reshape(NW * 3 * NSEG))
    means = out.reshape(2, NSEG)[:, :NSTRUCT].T
    return means


# speedup vs baseline: 8.9628x; 8.9628x over previous
import functools

import jax
import jax.numpy as jnp
from jax import lax
from jax.experimental import pallas as pl
from jax.experimental.pallas import tpu as pltpu
from jax.experimental.pallas import tpu_sc as plsc

N = 800000
NSTRUCT = 4000
NSEG = 4096
NC = 2
NS = 16
NW = NC * NS
NT = N // 128
TB = 10
NBLK = NT // TB
C1 = 3
C2 = 5
P = 8

_mesh = plsc.VectorSubcoreMesh(core_axis_name="c", subcore_axis_name="s")


@functools.partial(
    pl.kernel,
    out_type=jax.ShapeDtypeStruct((NW, 3 * NSEG), jnp.float32),
    mesh=_mesh,
    scratch_types=[
        pltpu.VMEM((C1 * TB * 1024,), jnp.float32),
        pltpu.VMEM((C2 * TB * 1024,), jnp.float32),
        pltpu.VMEM((TB * 128,), jnp.int32),
        pltpu.VMEM((3 * NSEG,), jnp.float32),
    ],
    compiler_params=pltpu.CompilerParams(needs_layout_passes=False),
)
def _partial_kernel(l1_hbm, l2_hbm, ids_hbm, part_hbm, l1b, l2b, idsb, acc):
    wid = lax.axis_index("s") * NC + lax.axis_index("c")

    zeros16 = jnp.zeros((16,), jnp.float32)

    def zero_body(i, _):
        acc[pl.ds(i * 16, 16)] = zeros16
        return _

    lax.fori_loop(0, 3 * NSEG // 16, zero_body, None)

    ones16 = jnp.ones((16,), jnp.float32)

    def tile_body(i, _):
        def group_body(g, _):
            off = g * 16
            acc1 = zeros16
            acc2 = zeros16
            for c in range(C1):
                for p in range(P):
                    v = l1b[pl.ds((c * TB + i) * 1024 + p * 128 + off, 16)]
                    acc1 = acc1 + v * v
            for c in range(C2):
                for p in range(P):
                    v = l2b[pl.ds((c * TB + i) * 1024 + p * 128 + off, 16)]
                    acc2 = acc2 + v * v
            ids_v = idsb[pl.ds(i * 128 + off, 16)]
            plsc.addupdate_scatter(acc, [ids_v], acc1)
            plsc.addupdate_scatter(acc, [ids_v + NSEG], acc2)
            plsc.addupdate_scatter(acc, [ids_v + 2 * NSEG], ones16)
            return _

        lax.fori_loop(0, 8, group_body, None)
        return _

    def block_body(i, _):
        blk = wid + i * NW
        t0 = blk * TB
        for c in range(C1):
            pltpu.sync_copy(
                l1_hbm.at[pl.ds(c * (NT * 1024) + t0 * 1024, TB * 1024)],
                l1b.at[pl.ds(c * (TB * 1024), TB * 1024)],
            )
        for c in range(C2):
            pltpu.sync_copy(
                l2_hbm.at[pl.ds(c * (NT * 1024) + t0 * 1024, TB * 1024)],
                l2b.at[pl.ds(c * (TB * 1024), TB * 1024)],
            )
        pltpu.sync_copy(ids_hbm.at[pl.ds(t0 * 128, TB * 128)], idsb)
        lax.fori_loop(0, TB, tile_body, None)
        return _

    nblk_w = (NBLK + NW - 1 - wid) // NW
    lax.fori_loop(0, nblk_w, block_body, None)

    pltpu.sync_copy(acc, part_hbm.at[wid])


_CHUNK = NSEG // NW


@functools.partial(
    pl.kernel,
    out_type=jax.ShapeDtypeStruct((2 * NSEG,), jnp.float32),
    mesh=_mesh,
    scratch_types=[
        pltpu.VMEM((_CHUNK,), jnp.float32),
        pltpu.VMEM((_CHUNK,), jnp.float32),
        pltpu.VMEM((_CHUNK,), jnp.float32),
        pltpu.VMEM((_CHUNK,), jnp.float32),
    ],
    compiler_params=pltpu.CompilerParams(needs_layout_passes=False),
)
def _finalize_kernel(part_hbm, out_hbm, s1, s2, cnt, tmp):
    wid = lax.axis_index("s") * NC + lax.axis_index("c")
    base = wid * _CHUNK
    zeros16 = jnp.zeros((16,), jnp.float32)

    def zero_body(i, _):
        s1[pl.ds(i * 16, 16)] = zeros16
        s2[pl.ds(i * 16, 16)] = zeros16
        cnt[pl.ds(i * 16, 16)] = zeros16
        return _

    lax.fori_loop(0, _CHUNK // 16, zero_body, None)

    def add_into(dst, i):
        def body(k, _):
            sl = pl.ds(k * 16, 16)
            dst[sl] = dst[sl] + tmp[sl]
            return _

        lax.fori_loop(0, _CHUNK // 16, body, None)

    def red_body(i, _):
        off = i * (3 * NSEG) + base
        pltpu.sync_copy(part_hbm.at[pl.ds(off, _CHUNK)], tmp)
        add_into(s1, i)
        pltpu.sync_copy(part_hbm.at[pl.ds(off + NSEG, _CHUNK)], tmp)
        add_into(s2, i)
        pltpu.sync_copy(part_hbm.at[pl.ds(off + 2 * NSEG, _CHUNK)], tmp)
        add_into(cnt, i)
        return _

    lax.fori_loop(0, NW, red_body, None)

    ones16 = jnp.ones((16,), jnp.float32)

    def div_body(k, _):
        sl = pl.ds(k * 16, 16)
        c = jnp.maximum(cnt[sl], ones16)
        s1[sl] = s1[sl] / c
        s2[sl] = s2[sl] / c
        return _

    lax.fori_loop(0, _CHUNK // 16, div_body, None)

    pltpu.sync_copy(s1, out_hbm.at[pl.ds(base, _CHUNK)])
    pltpu.sync_copy(s2, out_hbm.at[pl.ds(NSEG + base, _CHUNK)])


def _to_native_flat(x, ncomp):
    y = x.transpose(1, 0, 2)
    u = y.reshape(ncomp, NT, 128, P)
    z = u.transpose(0, 1, 3, 2)
    return z.reshape(ncomp * NT * P * 128)


def kernel(spex_l1, spex_l2, structure_ids):
    l1_flat = _to_native_flat(spex_l1, C1)
    l2_flat = _to_native_flat(spex_l2, C2)
    ids = structure_ids.astype(jnp.int32)
    part = _partial_kernel(l1_flat, l2_flat, ids)
    out = _finalize_kernel(part.reshape(NW * 3 * NSEG))
    means = out.reshape(2, NSEG)[:, :NSTRUCT].T
    return means


# speedup vs baseline: 19.0398x; 2.1243x over previous
import functools

import jax
import jax.numpy as jnp
from jax import lax
from jax.experimental import pallas as pl
from jax.experimental.pallas import tpu as pltpu
from jax.experimental.pallas import tpu_sc as plsc

N = 800000
NSTRUCT = 4000
NSEG = 4096
NC = 2
NS = 16
NW = NC * NS
NT = N // 128
TB = 2
NBLK = NT // TB
C1 = 3
C2 = 5
P = 8

_mesh = plsc.VectorSubcoreMesh(core_axis_name="c", subcore_axis_name="s")


@functools.partial(
    pl.kernel,
    out_type=jax.ShapeDtypeStruct((NW, 3 * NSEG), jnp.float32),
    mesh=_mesh,
    scratch_types=[
        pltpu.VMEM((C1 * TB * 1024,), jnp.float32),
        pltpu.VMEM((C1 * TB * 1024,), jnp.float32),
        pltpu.VMEM((C2 * TB * 1024,), jnp.float32),
        pltpu.VMEM((C2 * TB * 1024,), jnp.float32),
        pltpu.VMEM((TB * 128,), jnp.int32),
        pltpu.VMEM((TB * 128,), jnp.int32),
        pltpu.VMEM((3 * NSEG,), jnp.float32),
        pltpu.SemaphoreType.DMA,
        pltpu.SemaphoreType.DMA,
    ],
    compiler_params=pltpu.CompilerParams(needs_layout_passes=False),
)
def _partial_kernel(
    l1_hbm, l2_hbm, ids_hbm, part_hbm,
    l1b0, l1b1, l2b0, l2b1, idsb0, idsb1, acc, sem0, sem1,
):
    wid = lax.axis_index("s") * NC + lax.axis_index("c")
    nblk = (NBLK + NW - 1 - wid) // NW

    bufs = ((l1b0, l2b0, idsb0, sem0), (l1b1, l2b1, idsb1, sem1))

    zeros16 = jnp.zeros((16,), jnp.float32)
    ones16 = jnp.ones((16,), jnp.float32)

    def zero_body(i, _):
        acc[pl.ds(i * 16, 16)] = zeros16
        return _

    lax.fori_loop(0, 3 * NSEG // 16, zero_body, None)

    def fire(j, b):
        l1b, l2b, idsb, sem = bufs[b]
        t0 = (wid + j * NW) * TB
        for c in range(C1):
            pltpu.async_copy(
                l1_hbm.at[pl.ds(c * (NT * 1024) + t0 * 1024, TB * 1024)],
                l1b.at[pl.ds(c * (TB * 1024), TB * 1024)],
                sem,
            )
        for c in range(C2):
            pltpu.async_copy(
                l2_hbm.at[pl.ds(c * (NT * 1024) + t0 * 1024, TB * 1024)],
                l2b.at[pl.ds(c * (TB * 1024), TB * 1024)],
                sem,
            )
        pltpu.async_copy(ids_hbm.at[pl.ds(t0 * 128, TB * 128)], idsb, sem)

    def drain(b):
        l1b, l2b, idsb, sem = bufs[b]
        pltpu.make_async_copy(
            l1_hbm.at[pl.ds(0, C1 * TB * 1024)], l1b, sem
        ).wait()
        pltpu.make_async_copy(
            l2_hbm.at[pl.ds(0, C2 * TB * 1024)], l2b, sem
        ).wait()
        pltpu.make_async_copy(ids_hbm.at[pl.ds(0, TB * 128)], idsb, sem).wait()

    def compute(b):
        l1b, l2b, idsb, _ = bufs[b]

        def tile_body(t, _):
            def group_body(g, _):
                off = t * 1024 + g * 16
                aoff = t * 128 + g * 16
                acc1 = zeros16
                acc2 = zeros16
                for c in range(C1):
                    for p in range(P):
                        v = l1b[pl.ds((c * TB + t) * 1024 + g * 16 + p * 128, 16)]
                        acc1 = acc1 + v * v
                for c in range(C2):
                    for p in range(P):
                        v = l2b[pl.ds((c * TB + t) * 1024 + g * 16 + p * 128, 16)]
                        acc2 = acc2 + v * v
                ids_v = idsb[pl.ds(aoff, 16)]
                plsc.addupdate_scatter(acc, [ids_v], acc1)
                plsc.addupdate_scatter(acc, [ids_v + NSEG], acc2)
                plsc.addupdate_scatter(acc, [ids_v + 2 * NSEG], ones16)
                return _

            lax.fori_loop(0, 8, group_body, None)
            return _

        lax.fori_loop(0, TB, tile_body, None)

    fire(0, 0)

    def pair_body(k, _):
        i1 = 2 * k + 1

        @pl.when(i1 < nblk)
        def _fire1():
            fire(i1, 1)

        drain(0)
        compute(0)

        @pl.when(i1 + 1 < nblk)
        def _fire0():
            fire(i1 + 1, 0)

        @pl.when(i1 < nblk)
        def _do1():
            drain(1)
            compute(1)

        return _

    lax.fori_loop(0, (nblk + 1) // 2, pair_body, None)

    pltpu.sync_copy(acc, part_hbm.at[wid])


_CHUNK = NSEG // NW
_HALF = _CHUNK // 2


@functools.partial(
    pl.kernel,
    out_type=jax.ShapeDtypeStruct((2 * NSEG,), jnp.float32),
    mesh=_mesh,
    scratch_types=[
        pltpu.VMEM((NW * 3 * _HALF,), jnp.float32),
        pltpu.VMEM((_CHUNK,), jnp.float32),
        pltpu.VMEM((_CHUNK,), jnp.float32),
        pltpu.SemaphoreType.DMA,
    ],
    compiler_params=pltpu.CompilerParams(needs_layout_passes=False),
)
def _finalize_kernel(part_hbm, out_hbm, buf, outv0, outv1, sem):
    wid = lax.axis_index("s") * NC + lax.axis_index("c")
    base = wid * _CHUNK
    zeros16 = jnp.zeros((16,), jnp.float32)
    ones16 = jnp.ones((16,), jnp.float32)

    for h in range(2):
        def load_body(i, _):
            for r in range(3):
                pltpu.async_copy(
                    part_hbm.at[
                        pl.ds(i * (3 * NSEG) + r * NSEG + base + h * _HALF, _HALF)
                    ],
                    buf.at[pl.ds((i * 3 + r) * _HALF, _HALF)],
                    sem,
                )
            return _

        lax.fori_loop(0, NW, load_body, None)
        pltpu.make_async_copy(
            part_hbm.at[pl.ds(0, NW * 3 * _HALF)], buf, sem
        ).wait()

        for g in range(_HALF // 16):
            def red(r):
                def body(i, carry):
                    return carry + buf[pl.ds((i * 3 + r) * _HALF + g * 16, 16)]

                return lax.fori_loop(0, NW, body, zeros16)

            s1 = red(0)
            s2 = red(1)
            cnt = jnp.maximum(red(2), ones16)
            osl = pl.ds(h * _HALF + g * 16, 16)
            outv0[osl] = s1 / cnt
            outv1[osl] = s2 / cnt

    pltpu.sync_copy(outv0, out_hbm.at[pl.ds(base, _CHUNK)])
    pltpu.sync_copy(outv1, out_hbm.at[pl.ds(NSEG + base, _CHUNK)])


def _to_native_flat(x, ncomp):
    y = x.transpose(1, 0, 2)
    u = y.reshape(ncomp, NT, 128, P)
    z = u.transpose(0, 1, 3, 2)
    return z.reshape(ncomp * NT * P * 128)


def kernel(spex_l1, spex_l2, structure_ids):
    l1_flat = _to_native_flat(spex_l1, C1)
    l2_flat = _to_native_flat(spex_l2, C2)
    ids = structure_ids.astype(jnp.int32)
    part = _partial_kernel(l1_flat, l2_flat, ids)
    out = _finalize_kernel(part.reshape(NW * 3 * NSEG))
    means = out.reshape(2, NSEG)[:, :NSTRUCT].T
    return means


# speedup vs baseline: 19.2971x; 1.0135x over previous
import functools

import jax
import jax.numpy as jnp
from jax import lax
from jax.experimental import pallas as pl
from jax.experimental.pallas import tpu as pltpu
from jax.experimental.pallas import tpu_sc as plsc

N = 800000
NSTRUCT = 4000
NSEG = 4096
NC = 2
NS = 16
NW = NC * NS
NT = N // 128
TB = 5
NBLK = NT // TB
C1 = 3
C2 = 5
P = 8
ROWS = 3 * NSEG // 128

_mesh = plsc.VectorSubcoreMesh(core_axis_name="c", subcore_axis_name="s")


@functools.partial(
    pl.kernel,
    out_type=jax.ShapeDtypeStruct((NC, ROWS, 128), jnp.float32),
    mesh=_mesh,
    scratch_types=[
        pltpu.VMEM((C1 * TB * 1024,), jnp.float32),
        pltpu.VMEM((C1 * TB * 1024,), jnp.float32),
        pltpu.VMEM((C2 * TB * 1024,), jnp.float32),
        pltpu.VMEM((C2 * TB * 1024,), jnp.float32),
        pltpu.VMEM((TB * 128,), jnp.int32),
        pltpu.VMEM((TB * 128,), jnp.int32),
        pltpu.VMEM((ROWS, 128), jnp.float32),
        pltpu.VMEM((ROWS,), jnp.int32),
        pltpu.VMEM_SHARED((ROWS, 128), jnp.float32),
        pltpu.SemaphoreType.DMA,
        pltpu.SemaphoreType.DMA,
    ],
    compiler_params=pltpu.CompilerParams(needs_layout_passes=False),
)
def _partial_kernel(
    l1_hbm, l2_hbm, ids_hbm, part_hbm,
    l1b0, l1b1, l2b0, l2b1, idsb0, idsb1, acc, rowidx, shared, sem0, sem1,
):
    cid = lax.axis_index("c")
    sid = lax.axis_index("s")
    wid = sid * NC + cid
    nblk = (NBLK + NW - 1 - wid) // NW

    bufs = ((l1b0, l2b0, idsb0, sem0), (l1b1, l2b1, idsb1, sem1))

    zeros16 = jnp.zeros((16,), jnp.float32)
    ones16 = jnp.ones((16,), jnp.float32)
    iota16 = lax.iota(jnp.int32, 16)

    def zero_body(i, _):
        acc[i, pl.ds(0, 16)] = zeros16
        for g in range(1, 8):
            acc[i, pl.ds(g * 16, 16)] = zeros16
        return _

    lax.fori_loop(0, ROWS, zero_body, None)

    def idx_body(i, _):
        rowidx[pl.ds(i * 16, 16)] = iota16 + i * 16
        return _

    lax.fori_loop(0, ROWS // 16, idx_body, None)

    def fire(j, b):
        l1b, l2b, idsb, sem = bufs[b]
        t0 = (wid + j * NW) * TB
        for c in range(C1):
            pltpu.async_copy(
                l1_hbm.at[pl.ds(c * (NT * 1024) + t0 * 1024, TB * 1024)],
                l1b.at[pl.ds(c * (TB * 1024), TB * 1024)],
                sem,
            )
        for c in range(C2):
            pltpu.async_copy(
                l2_hbm.at[pl.ds(c * (NT * 1024) + t0 * 1024, TB * 1024)],
                l2b.at[pl.ds(c * (TB * 1024), TB * 1024)],
                sem,
            )
        pltpu.async_copy(ids_hbm.at[pl.ds(t0 * 128, TB * 128)], idsb, sem)

    def drain(b):
        l1b, l2b, idsb, sem = bufs[b]
        pltpu.make_async_copy(
            l1_hbm.at[pl.ds(0, C1 * TB * 1024)], l1b, sem
        ).wait()
        pltpu.make_async_copy(
            l2_hbm.at[pl.ds(0, C2 * TB * 1024)], l2b, sem
        ).wait()
        pltpu.make_async_copy(ids_hbm.at[pl.ds(0, TB * 128)], idsb, sem).wait()

    def compute(b):
        l1b, l2b, idsb, _ = bufs[b]

        def tile_body(t, _):
            def group_body(g, _):
                acc1 = zeros16
                acc2 = zeros16
                for c in range(C1):
                    for p in range(P):
                        v = l1b[pl.ds((c * TB + t) * 1024 + g * 16 + p * 128, 16)]
                        acc1 = acc1 + v * v
                for c in range(C2):
                    for p in range(P):
                        v = l2b[pl.ds((c * TB + t) * 1024 + g * 16 + p * 128, 16)]
                        acc2 = acc2 + v * v
                ids_v = idsb[pl.ds(t * 128 + g * 16, 16)]
                row = lax.shift_right_logical(ids_v, 7)
                col = lax.bitwise_and(ids_v, 127)
                plsc.addupdate_scatter(acc, [row, col], acc1)
                plsc.addupdate_scatter(acc, [row + 32, col], acc2)
                plsc.addupdate_scatter(acc, [row + 64, col], ones16)
                return _

            lax.fori_loop(0, 8, group_body, None)
            return _

        lax.fori_loop(0, TB, tile_body, None)

    fire(0, 0)

    def pair_body(k, _):
        i1 = 2 * k + 1

        @pl.when(i1 < nblk)
        def _fire1():
            fire(i1, 1)

        drain(0)
        compute(0)

        @pl.when(i1 + 1 < nblk)
        def _fire0():
            fire(i1 + 1, 0)

        @pl.when(i1 < nblk)
        def _do1():
            drain(1)
            compute(1)

        return _

    lax.fori_loop(0, (nblk + 1) // 2, pair_body, None)

    @pl.when(sid == 0)
    def _init_shared():
        pltpu.sync_copy(acc, shared)

    plsc.subcore_barrier()

    @pl.when(sid != 0)
    def _add_shared():
        pltpu.sync_copy(acc, shared.at[rowidx], add=True)

    plsc.subcore_barrier()

    @pl.when(sid == 0)
    def _write_out():
        pltpu.sync_copy(shared, part_hbm.at[cid])


_CHUNK = NSEG // NW


@functools.partial(
    pl.kernel,
    out_type=jax.ShapeDtypeStruct((2 * NSEG,), jnp.float32),
    mesh=_mesh,
    scratch_types=[
        pltpu.VMEM((NC * 3 * _CHUNK,), jnp.float32),
        pltpu.VMEM((_CHUNK,), jnp.float32),
        pltpu.VMEM((_CHUNK,), jnp.float32),
        pltpu.SemaphoreType.DMA,
    ],
    compiler_params=pltpu.CompilerParams(needs_layout_passes=False),
)
def _finalize_kernel(part_hbm, out_hbm, buf, outv0, outv1, sem):
    wid = lax.axis_index("s") * NC + lax.axis_index("c")
    base = wid * _CHUNK
    ones16 = jnp.ones((16,), jnp.float32)

    for i in range(NC):
        for r in range(3):
            pltpu.async_copy(
                part_hbm.at[pl.ds(i * (3 * NSEG) + r * NSEG + base, _CHUNK)],
                buf.at[pl.ds((i * 3 + r) * _CHUNK, _CHUNK)],
                sem,
            )
    pltpu.make_async_copy(
        part_hbm.at[pl.ds(0, NC * 3 * _CHUNK)], buf, sem
    ).wait()

    for g in range(_CHUNK // 16):
        def red(r):
            a = buf[pl.ds(r * _CHUNK + g * 16, 16)]
            b = buf[pl.ds((3 + r) * _CHUNK + g * 16, 16)]
            return a + b

        s1 = red(0)
        s2 = red(1)
        cnt = jnp.maximum(red(2), ones16)
        osl = pl.ds(g * 16, 16)
        outv0[osl] = s1 / cnt
        outv1[osl] = s2 / cnt

    pltpu.sync_copy(outv0, out_hbm.at[pl.ds(base, _CHUNK)])
    pltpu.sync_copy(outv1, out_hbm.at[pl.ds(NSEG + base, _CHUNK)])


def _to_native_flat(x, ncomp):
    y = x.transpose(1, 0, 2)
    u = y.reshape(ncomp, NT, 128, P)
    z = u.transpose(0, 1, 3, 2)
    return z.reshape(ncomp * NT * P * 128)


def kernel(spex_l1, spex_l2, structure_ids):
    l1_flat = _to_native_flat(spex_l1, C1)
    l2_flat = _to_native_flat(spex_l2, C2)
    ids = structure_ids.astype(jnp.int32)
    part = _partial_kernel(l1_flat, l2_flat, ids)
    out = _finalize_kernel(part.reshape(NC * ROWS * 128))
    means = out.reshape(2, NSEG)[:, :NSTRUCT].T
    return means


# speedup vs baseline: 19.9849x; 1.0356x over previous
import functools

import jax
import jax.numpy as jnp
from jax import lax
from jax.experimental import pallas as pl
from jax.experimental.pallas import tpu as pltpu
from jax.experimental.pallas import tpu_sc as plsc

N = 800000
NSTRUCT = 4000
NSEG = 4096
NC = 2
NS = 16
NW = NC * NS
NT = N // 128
TB = 5
NBLK = NT // TB
C1 = 3
C2 = 5
P = 8
ROWS = 3 * NSEG // 128

_mesh = plsc.VectorSubcoreMesh(core_axis_name="c", subcore_axis_name="s")


@functools.partial(
    pl.kernel,
    out_type=jax.ShapeDtypeStruct((NC, ROWS, 128), jnp.float32),
    mesh=_mesh,
    scratch_types=[
        pltpu.VMEM((C1 * TB * 1024,), jnp.float32),
        pltpu.VMEM((C1 * TB * 1024,), jnp.float32),
        pltpu.VMEM((C2 * TB * 1024,), jnp.float32),
        pltpu.VMEM((C2 * TB * 1024,), jnp.float32),
        pltpu.VMEM((TB * 128,), jnp.int32),
        pltpu.VMEM((TB * 128,), jnp.int32),
        pltpu.VMEM((ROWS, 128), jnp.float32),
        pltpu.VMEM((ROWS,), jnp.int32),
        pltpu.VMEM_SHARED((ROWS, 128), jnp.float32),
        pltpu.SemaphoreType.DMA,
        pltpu.SemaphoreType.DMA,
    ],
    compiler_params=pltpu.CompilerParams(needs_layout_passes=False),
)
def _partial_kernel(
    l1_hbm, l2_hbm, ids_hbm, part_hbm,
    l1b0, l1b1, l2b0, l2b1, idsb0, idsb1, acc, rowidx, shared, sem0, sem1,
):
    cid = lax.axis_index("c")
    sid = lax.axis_index("s")
    wid = sid * NC + cid
    nblk = (NBLK + NW - 1 - wid) // NW

    bufs = ((l1b0, l2b0, idsb0, sem0), (l1b1, l2b1, idsb1, sem1))

    zeros16 = jnp.zeros((16,), jnp.float32)
    ones16 = jnp.ones((16,), jnp.float32)
    iota16 = lax.iota(jnp.int32, 16)

    def zero_body(i, _):
        acc[i, pl.ds(0, 16)] = zeros16
        for g in range(1, 8):
            acc[i, pl.ds(g * 16, 16)] = zeros16
        return _

    lax.fori_loop(0, ROWS, zero_body, None)

    def idx_body(i, _):
        rowidx[pl.ds(i * 16, 16)] = iota16 + i * 16
        return _

    lax.fori_loop(0, ROWS // 16, idx_body, None)

    def fire(j, b):
        l1b, l2b, idsb, sem = bufs[b]
        t0 = (wid + j * NW) * TB
        for c in range(C1):
            pltpu.async_copy(
                l1_hbm.at[pl.ds(c * (NT * 1024) + t0 * 1024, TB * 1024)],
                l1b.at[pl.ds(c * (TB * 1024), TB * 1024)],
                sem,
            )
        for c in range(C2):
            pltpu.async_copy(
                l2_hbm.at[pl.ds(c * (NT * 1024) + t0 * 1024, TB * 1024)],
                l2b.at[pl.ds(c * (TB * 1024), TB * 1024)],
                sem,
            )
        pltpu.async_copy(ids_hbm.at[pl.ds(t0 * 128, TB * 128)], idsb, sem)

    def drain(b):
        l1b, l2b, idsb, sem = bufs[b]
        pltpu.make_async_copy(
            l1_hbm.at[pl.ds(0, C1 * TB * 1024)], l1b, sem
        ).wait()
        pltpu.make_async_copy(
            l2_hbm.at[pl.ds(0, C2 * TB * 1024)], l2b, sem
        ).wait()
        pltpu.make_async_copy(ids_hbm.at[pl.ds(0, TB * 128)], idsb, sem).wait()

    def compute(b):
        l1b, l2b, idsb, _ = bufs[b]

        def tile_body(t, _):
            def group_body(g, _):
                sq1 = []
                for c in range(C1):
                    for p in range(P):
                        v = l1b[pl.ds((c * TB + t) * 1024 + g * 16 + p * 128, 16)]
                        sq1.append(v * v)
                sq2 = []
                for c in range(C2):
                    for p in range(P):
                        v = l2b[pl.ds((c * TB + t) * 1024 + g * 16 + p * 128, 16)]
                        sq2.append(v * v)

                def tree_sum(vals):
                    while len(vals) > 1:
                        nxt = [
                            vals[i] + vals[i + 1]
                            for i in range(0, len(vals) - 1, 2)
                        ]
                        if len(vals) % 2:
                            nxt.append(vals[-1])
                        vals = nxt
                    return vals[0]

                acc1 = tree_sum(sq1)
                acc2 = tree_sum(sq2)
                ids_v = idsb[pl.ds(t * 128 + g * 16, 16)]
                row = lax.shift_right_logical(ids_v, 7)
                col = lax.bitwise_and(ids_v, 127)
                plsc.addupdate_scatter(acc, [row, col], acc1)
                plsc.addupdate_scatter(acc, [row + 32, col], acc2)
                plsc.addupdate_scatter(acc, [row + 64, col], ones16)
                return _

            lax.fori_loop(0, 8, group_body, None)
            return _

        lax.fori_loop(0, TB, tile_body, None)

    fire(0, 0)

    def pair_body(k, _):
        i1 = 2 * k + 1

        @pl.when(i1 < nblk)
        def _fire1():
            fire(i1, 1)

        drain(0)
        compute(0)

        @pl.when(i1 + 1 < nblk)
        def _fire0():
            fire(i1 + 1, 0)

        @pl.when(i1 < nblk)
        def _do1():
            drain(1)
            compute(1)

        return _

    lax.fori_loop(0, (nblk + 1) // 2, pair_body, None)

    @pl.when(sid == 0)
    def _init_shared():
        pltpu.sync_copy(acc, shared)

    plsc.subcore_barrier()

    @pl.when(sid != 0)
    def _add_shared():
        pltpu.sync_copy(acc, shared.at[rowidx], add=True)

    plsc.subcore_barrier()

    @pl.when(sid == 0)
    def _write_out():
        pltpu.sync_copy(shared, part_hbm.at[cid])


_CHUNK = NSEG // NW


@functools.partial(
    pl.kernel,
    out_type=jax.ShapeDtypeStruct((2 * NSEG,), jnp.float32),
    mesh=_mesh,
    scratch_types=[
        pltpu.VMEM((NC * 3 * _CHUNK,), jnp.float32),
        pltpu.VMEM((_CHUNK,), jnp.float32),
        pltpu.VMEM((_CHUNK,), jnp.float32),
        pltpu.SemaphoreType.DMA,
    ],
    compiler_params=pltpu.CompilerParams(needs_layout_passes=False),
)
def _finalize_kernel(part_hbm, out_hbm, buf, outv0, outv1, sem):
    wid = lax.axis_index("s") * NC + lax.axis_index("c")
    base = wid * _CHUNK
    ones16 = jnp.ones((16,), jnp.float32)

    for i in range(NC):
        for r in range(3):
            pltpu.async_copy(
                part_hbm.at[pl.ds(i * (3 * NSEG) + r * NSEG + base, _CHUNK)],
                buf.at[pl.ds((i * 3 + r) * _CHUNK, _CHUNK)],
                sem,
            )
    pltpu.make_async_copy(
        part_hbm.at[pl.ds(0, NC * 3 * _CHUNK)], buf, sem
    ).wait()

    for g in range(_CHUNK // 16):
        def red(r):
            a = buf[pl.ds(r * _CHUNK + g * 16, 16)]
            b = buf[pl.ds((3 + r) * _CHUNK + g * 16, 16)]
            return a + b

        s1 = red(0)
        s2 = red(1)
        cnt = jnp.maximum(red(2), ones16)
        osl = pl.ds(g * 16, 16)
        outv0[osl] = s1 / cnt
        outv1[osl] = s2 / cnt

    pltpu.sync_copy(outv0, out_hbm.at[pl.ds(base, _CHUNK)])
    pltpu.sync_copy(outv1, out_hbm.at[pl.ds(NSEG + base, _CHUNK)])


def _to_native_flat(x, ncomp):
    y = x.transpose(1, 0, 2)
    u = y.reshape(ncomp, NT, 128, P)
    z = u.transpose(0, 1, 3, 2)
    return z.reshape(ncomp * NT * P * 128)


def kernel(spex_l1, spex_l2, structure_ids):
    l1_flat = _to_native_flat(spex_l1, C1)
    l2_flat = _to_native_flat(spex_l2, C2)
    ids = structure_ids.astype(jnp.int32)
    part = _partial_kernel(l1_flat, l2_flat, ids)
    out = _finalize_kernel(part.reshape(NC * ROWS * 128))
    means = out.reshape(2, NSEG)[:, :NSTRUCT].T
    return means


# speedup vs baseline: 23.3950x; 1.1706x over previous
import functools

import jax
import jax.numpy as jnp
from jax import lax
from jax.experimental import pallas as pl
from jax.experimental.pallas import tpu as pltpu
from jax.experimental.pallas import tpu_sc as plsc

N = 800000
NSTRUCT = 4000
NSEG = 4096
NC = 2
NS = 16
NW = NC * NS
NT = N // 128
TB = 5
NBLK = NT // TB
C1 = 3
C2 = 5
P = 8
ROWS = 3 * NSEG // 128

_mesh = plsc.VectorSubcoreMesh(core_axis_name="c", subcore_axis_name="s")


@functools.partial(
    pl.kernel,
    out_type=jax.ShapeDtypeStruct((NC, ROWS, 128), jnp.float32),
    mesh=_mesh,
    scratch_types=[
        pltpu.VMEM((C1 * TB * 1024,), jnp.float32),
        pltpu.VMEM((C1 * TB * 1024,), jnp.float32),
        pltpu.VMEM((C2 * TB * 1024,), jnp.float32),
        pltpu.VMEM((C2 * TB * 1024,), jnp.float32),
        pltpu.VMEM((TB * 128,), jnp.int32),
        pltpu.VMEM((TB * 128,), jnp.int32),
        pltpu.VMEM((ROWS, 128), jnp.float32),
        pltpu.VMEM((ROWS,), jnp.int32),
        pltpu.VMEM_SHARED((ROWS, 128), jnp.float32),
        pltpu.SemaphoreType.DMA,
        pltpu.SemaphoreType.DMA,
    ],
    compiler_params=pltpu.CompilerParams(needs_layout_passes=False),
)
def _partial_kernel(
    l1_hbm, l2_hbm, ids_hbm, part_hbm,
    l1b0, l1b1, l2b0, l2b1, idsb0, idsb1, acc, rowidx, shared, sem0, sem1,
):
    cid = lax.axis_index("c")
    sid = lax.axis_index("s")
    wid = sid * NC + cid
    nblk = (NBLK + NW - 1 - wid) // NW

    bufs = ((l1b0, l2b0, idsb0, sem0), (l1b1, l2b1, idsb1, sem1))

    zeros16 = jnp.zeros((16,), jnp.float32)
    ones16 = jnp.ones((16,), jnp.float32)
    iota16 = lax.iota(jnp.int32, 16)

    def zero_body(i, _):
        acc[i, pl.ds(0, 16)] = zeros16
        for g in range(1, 8):
            acc[i, pl.ds(g * 16, 16)] = zeros16
        return _

    lax.fori_loop(0, ROWS, zero_body, None)

    def idx_body(i, _):
        rowidx[pl.ds(i * 16, 16)] = iota16 + i * 16
        return _

    lax.fori_loop(0, ROWS // 16, idx_body, None)

    def fire(j, b):
        l1b, l2b, idsb, sem = bufs[b]
        t0 = (wid + j * NW) * TB
        for c in range(C1):
            pltpu.async_copy(
                l1_hbm.at[pl.ds(c * (NT * 1024) + t0 * 1024, TB * 1024)],
                l1b.at[pl.ds(c * (TB * 1024), TB * 1024)],
                sem,
            )
        for c in range(C2):
            pltpu.async_copy(
                l2_hbm.at[pl.ds(c * (NT * 1024) + t0 * 1024, TB * 1024)],
                l2b.at[pl.ds(c * (TB * 1024), TB * 1024)],
                sem,
            )
        pltpu.async_copy(ids_hbm.at[pl.ds(t0 * 128, TB * 128)], idsb, sem)

    def drain(b):
        l1b, l2b, idsb, sem = bufs[b]
        pltpu.make_async_copy(
            l1_hbm.at[pl.ds(0, C1 * TB * 1024)], l1b, sem
        ).wait()
        pltpu.make_async_copy(
            l2_hbm.at[pl.ds(0, C2 * TB * 1024)], l2b, sem
        ).wait()
        pltpu.make_async_copy(ids_hbm.at[pl.ds(0, TB * 128)], idsb, sem).wait()

    def compute(b):
        l1b, l2b, idsb, _ = bufs[b]

        def tile_body(t, _):
            ids_head = idsb[pl.ds(t * 128, 16)]
            ids_tail = idsb[pl.ds(t * 128 + 112, 16)]
            id_first = ids_head[0]
            uniform = id_first == ids_tail[15]

            def tree_sum(vals):
                while len(vals) > 1:
                    nxt = [
                        vals[i] + vals[i + 1]
                        for i in range(0, len(vals) - 1, 2)
                    ]
                    if len(vals) % 2:
                        nxt.append(vals[-1])
                    vals = nxt
                return vals[0]

            def group_sums(g):
                sq1 = []
                for c in range(C1):
                    for p in range(P):
                        v = l1b[pl.ds((c * TB + t) * 1024 + g * 16 + p * 128, 16)]
                        sq1.append(v * v)
                sq2 = []
                for c in range(C2):
                    for p in range(P):
                        v = l2b[pl.ds((c * TB + t) * 1024 + g * 16 + p * 128, 16)]
                        sq2.append(v * v)
                return tree_sum(sq1), tree_sum(sq2)

            def uniform_body(g, carry):
                vq1, vq2 = carry
                acc1, acc2 = group_sums(g)
                return vq1 + acc1, vq2 + acc2

            def scatter_body(g, _):
                acc1, acc2 = group_sums(g)
                ids_v = idsb[pl.ds(t * 128 + g * 16, 16)]
                row = lax.shift_right_logical(ids_v, 7)
                col = lax.bitwise_and(ids_v, 127)
                plsc.addupdate_scatter(acc, [row, col], acc1)
                plsc.addupdate_scatter(acc, [row + 32, col], acc2)
                plsc.addupdate_scatter(acc, [row + 64, col], ones16)
                return _

            @pl.when(uniform)
            def _fast():
                vq1, vq2 = lax.fori_loop(
                    0, 8, uniform_body, (zeros16, zeros16)
                )
                s1 = jnp.sum(vq1)
                s2 = jnp.sum(vq2)
                rowv = (
                    lax.shift_right_logical(ids_head, 7)
                    + jnp.minimum(iota16, 2) * 32
                )
                colv = lax.bitwise_and(ids_head, 127)
                valv = jnp.where(
                    iota16 == 0, s1, jnp.where(iota16 == 1, s2, 128.0)
                )
                plsc.addupdate_scatter(acc, [rowv, colv], valv, mask=iota16 < 3)

            @pl.when(jnp.logical_not(uniform))
            def _slow():
                lax.fori_loop(0, 8, scatter_body, None)

            return _

        lax.fori_loop(0, TB, tile_body, None)

    fire(0, 0)

    def pair_body(k, _):
        i1 = 2 * k + 1

        @pl.when(i1 < nblk)
        def _fire1():
            fire(i1, 1)

        drain(0)
        compute(0)

        @pl.when(i1 + 1 < nblk)
        def _fire0():
            fire(i1 + 1, 0)

        @pl.when(i1 < nblk)
        def _do1():
            drain(1)
            compute(1)

        return _

    lax.fori_loop(0, (nblk + 1) // 2, pair_body, None)

    @pl.when(sid == 0)
    def _init_shared():
        pltpu.sync_copy(acc, shared)

    plsc.subcore_barrier()

    @pl.when(sid != 0)
    def _add_shared():
        pltpu.sync_copy(acc, shared.at[rowidx], add=True)

    plsc.subcore_barrier()

    @pl.when(sid == 0)
    def _write_out():
        pltpu.sync_copy(shared, part_hbm.at[cid])


_CHUNK = NSEG // NW


@functools.partial(
    pl.kernel,
    out_type=jax.ShapeDtypeStruct((2 * NSEG,), jnp.float32),
    mesh=_mesh,
    scratch_types=[
        pltpu.VMEM((NC * 3 * _CHUNK,), jnp.float32),
        pltpu.VMEM((_CHUNK,), jnp.float32),
        pltpu.VMEM((_CHUNK,), jnp.float32),
        pltpu.SemaphoreType.DMA,
    ],
    compiler_params=pltpu.CompilerParams(needs_layout_passes=False),
)
def _finalize_kernel(part_hbm, out_hbm, buf, outv0, outv1, sem):
    wid = lax.axis_index("s") * NC + lax.axis_index("c")
    base = wid * _CHUNK
    ones16 = jnp.ones((16,), jnp.float32)

    for i in range(NC):
        for r in range(3):
            pltpu.async_copy(
                part_hbm.at[pl.ds(i * (3 * NSEG) + r * NSEG + base, _CHUNK)],
                buf.at[pl.ds((i * 3 + r) * _CHUNK, _CHUNK)],
                sem,
            )
    pltpu.make_async_copy(
        part_hbm.at[pl.ds(0, NC * 3 * _CHUNK)], buf, sem
    ).wait()

    for g in range(_CHUNK // 16):
        def red(r):
            a = buf[pl.ds(r * _CHUNK + g * 16, 16)]
            b = buf[pl.ds((3 + r) * _CHUNK + g * 16, 16)]
            return a + b

        s1 = red(0)
        s2 = red(1)
        cnt = jnp.maximum(red(2), ones16)
        osl = pl.ds(g * 16, 16)
        outv0[osl] = s1 / cnt
        outv1[osl] = s2 / cnt

    pltpu.sync_copy(outv0, out_hbm.at[pl.ds(base, _CHUNK)])
    pltpu.sync_copy(outv1, out_hbm.at[pl.ds(NSEG + base, _CHUNK)])


def _to_native_flat(x, ncomp):
    y = x.transpose(1, 0, 2)
    u = y.reshape(ncomp, NT, 128, P)
    z = u.transpose(0, 1, 3, 2)
    return z.reshape(ncomp * NT * P * 128)


def kernel(spex_l1, spex_l2, structure_ids):
    l1_flat = _to_native_flat(spex_l1, C1)
    l2_flat = _to_native_flat(spex_l2, C2)
    ids = structure_ids.astype(jnp.int32)
    part = _partial_kernel(l1_flat, l2_flat, ids)
    out = _finalize_kernel(part.reshape(NC * ROWS * 128))
    means = out.reshape(2, NSEG)[:, :NSTRUCT].T
    return means


# speedup vs baseline: 29.6697x; 1.2682x over previous
import functools

import jax
import jax.numpy as jnp
from jax import lax
from jax.experimental import pallas as pl
from jax.experimental.pallas import tpu as pltpu
from jax.experimental.pallas import tpu_sc as plsc

N = 800000
NSTRUCT = 4000
NSEG = 4096
NC = 2
NS = 16
NW = NC * NS
NT = N // 128
TB = 5
NBLK = NT // TB
C1 = 3
C2 = 5
P = 8
ROWS = 3 * NSEG // 128

_mesh = plsc.VectorSubcoreMesh(core_axis_name="c", subcore_axis_name="s")


@functools.partial(
    pl.kernel,
    out_type=jax.ShapeDtypeStruct((NC, ROWS, 128), jnp.float32),
    mesh=_mesh,
    scratch_types=[
        pltpu.VMEM((C1 * TB * 1024,), jnp.float32),
        pltpu.VMEM((C1 * TB * 1024,), jnp.float32),
        pltpu.VMEM((C2 * TB * 1024,), jnp.float32),
        pltpu.VMEM((C2 * TB * 1024,), jnp.float32),
        pltpu.VMEM((TB * 128,), jnp.int32),
        pltpu.VMEM((TB * 128,), jnp.int32),
        pltpu.VMEM((ROWS, 128), jnp.float32),
        pltpu.VMEM((ROWS,), jnp.int32),
        pltpu.VMEM_SHARED((ROWS, 128), jnp.float32),
        pltpu.SemaphoreType.DMA,
        pltpu.SemaphoreType.DMA,
    ],
    compiler_params=pltpu.CompilerParams(needs_layout_passes=False),
)
def _partial_kernel(
    l1_hbm, l2_hbm, ids_hbm, part_hbm,
    l1b0, l1b1, l2b0, l2b1, idsb0, idsb1, acc, rowidx, shared, sem0, sem1,
):
    cid = lax.axis_index("c")
    sid = lax.axis_index("s")
    wid = sid * NC + cid
    nblk = (NBLK + NW - 1 - wid) // NW

    bufs = ((l1b0, l2b0, idsb0, sem0), (l1b1, l2b1, idsb1, sem1))

    zeros16 = jnp.zeros((16,), jnp.float32)
    ones16 = jnp.ones((16,), jnp.float32)
    iota16 = lax.iota(jnp.int32, 16)

    def zero_body(i, _):
        acc[i, pl.ds(0, 16)] = zeros16
        for g in range(1, 8):
            acc[i, pl.ds(g * 16, 16)] = zeros16
        return _

    lax.fori_loop(0, ROWS, zero_body, None)

    def idx_body(i, _):
        rowidx[pl.ds(i * 16, 16)] = iota16 + i * 16
        return _

    lax.fori_loop(0, ROWS // 16, idx_body, None)

    def fire(j, b):
        l1b, l2b, idsb, sem = bufs[b]
        t0 = (wid + j * NW) * TB
        for c in range(C1):
            pltpu.async_copy(
                l1_hbm.at[pl.ds(c * (NT * 1024) + t0 * 1024, TB * 1024)],
                l1b.at[pl.ds(c * (TB * 1024), TB * 1024)],
                sem,
            )
        for c in range(C2):
            pltpu.async_copy(
                l2_hbm.at[pl.ds(c * (NT * 1024) + t0 * 1024, TB * 1024)],
                l2b.at[pl.ds(c * (TB * 1024), TB * 1024)],
                sem,
            )
        pltpu.async_copy(ids_hbm.at[pl.ds(t0 * 128, TB * 128)], idsb, sem)

    def drain(b):
        l1b, l2b, idsb, sem = bufs[b]
        pltpu.make_async_copy(
            l1_hbm.at[pl.ds(0, C1 * TB * 1024)], l1b, sem
        ).wait()
        pltpu.make_async_copy(
            l2_hbm.at[pl.ds(0, C2 * TB * 1024)], l2b, sem
        ).wait()
        pltpu.make_async_copy(ids_hbm.at[pl.ds(0, TB * 128)], idsb, sem).wait()

    def compute(b):
        l1b, l2b, idsb, _ = bufs[b]

        def tile_body(t, _):
            ids_head = idsb[pl.ds(t * 128, 16)]
            ids_tail = idsb[pl.ds(t * 128 + 112, 16)]
            id_first = ids_head[0]
            uniform = id_first == ids_tail[15]

            def tree_sum(vals):
                while len(vals) > 1:
                    nxt = [
                        vals[i] + vals[i + 1]
                        for i in range(0, len(vals) - 1, 2)
                    ]
                    if len(vals) % 2:
                        nxt.append(vals[-1])
                    vals = nxt
                return vals[0]

            def group_sums(g):
                sq1 = []
                for c in range(C1):
                    for p in range(P):
                        v = l1b[pl.ds((c * TB + t) * 1024 + g * 16 + p * 128, 16)]
                        sq1.append(v * v)
                sq2 = []
                for c in range(C2):
                    for p in range(P):
                        v = l2b[pl.ds((c * TB + t) * 1024 + g * 16 + p * 128, 16)]
                        sq2.append(v * v)
                return tree_sum(sq1), tree_sum(sq2)

            def uniform_body(g, carry):
                vq1, vq2 = carry
                acc1, acc2 = group_sums(g)
                return vq1 + acc1, vq2 + acc2

            def scatter_body(g, _):
                acc1, acc2 = group_sums(g)
                ids_v = idsb[pl.ds(t * 128 + g * 16, 16)]
                g_uniform = ids_v[0] == ids_v[15]

                @pl.when(g_uniform)
                def _g_fast():
                    s1 = jnp.sum(acc1)
                    s2 = jnp.sum(acc2)
                    rowv = (
                        lax.shift_right_logical(ids_v, 7)
                        + jnp.minimum(iota16, 2) * 32
                    )
                    colv = lax.bitwise_and(ids_v, 127)
                    valv = jnp.where(
                        iota16 == 0, s1, jnp.where(iota16 == 1, s2, 16.0)
                    )
                    plsc.addupdate_scatter(
                        acc, [rowv, colv], valv, mask=iota16 < 3
                    )

                @pl.when(jnp.logical_not(g_uniform))
                def _g_slow():
                    row = lax.shift_right_logical(ids_v, 7)
                    col = lax.bitwise_and(ids_v, 127)
                    plsc.addupdate_scatter(acc, [row, col], acc1)
                    plsc.addupdate_scatter(acc, [row + 32, col], acc2)
                    plsc.addupdate_scatter(acc, [row + 64, col], ones16)

                return _

            @pl.when(uniform)
            def _fast():
                vq1, vq2 = lax.fori_loop(
                    0, 8, uniform_body, (zeros16, zeros16)
                )
                s1 = jnp.sum(vq1)
                s2 = jnp.sum(vq2)
                rowv = (
                    lax.shift_right_logical(ids_head, 7)
                    + jnp.minimum(iota16, 2) * 32
                )
                colv = lax.bitwise_and(ids_head, 127)
                valv = jnp.where(
                    iota16 == 0, s1, jnp.where(iota16 == 1, s2, 128.0)
                )
                plsc.addupdate_scatter(acc, [rowv, colv], valv, mask=iota16 < 3)

            @pl.when(jnp.logical_not(uniform))
            def _slow():
                lax.fori_loop(0, 8, scatter_body, None)

            return _

        lax.fori_loop(0, TB, tile_body, None)

    fire(0, 0)

    def pair_body(k, _):
        i1 = 2 * k + 1

        @pl.when(i1 < nblk)
        def _fire1():
            fire(i1, 1)

        drain(0)
        compute(0)

        @pl.when(i1 + 1 < nblk)
        def _fire0():
            fire(i1 + 1, 0)

        @pl.when(i1 < nblk)
        def _do1():
            drain(1)
            compute(1)

        return _

    lax.fori_loop(0, (nblk + 1) // 2, pair_body, None)

    @pl.when(sid == 0)
    def _init_shared():
        pltpu.sync_copy(acc, shared)

    plsc.subcore_barrier()

    @pl.when(sid != 0)
    def _add_shared():
        pltpu.sync_copy(acc, shared.at[rowidx], add=True)

    plsc.subcore_barrier()

    @pl.when(sid == 0)
    def _write_out():
        pltpu.sync_copy(shared, part_hbm.at[cid])


_CHUNK = NSEG // NW


@functools.partial(
    pl.kernel,
    out_type=jax.ShapeDtypeStruct((2 * NSEG,), jnp.float32),
    mesh=_mesh,
    scratch_types=[
        pltpu.VMEM((NC * 3 * _CHUNK,), jnp.float32),
        pltpu.VMEM((_CHUNK,), jnp.float32),
        pltpu.VMEM((_CHUNK,), jnp.float32),
        pltpu.SemaphoreType.DMA,
    ],
    compiler_params=pltpu.CompilerParams(needs_layout_passes=False),
)
def _finalize_kernel(part_hbm, out_hbm, buf, outv0, outv1, sem):
    wid = lax.axis_index("s") * NC + lax.axis_index("c")
    base = wid * _CHUNK
    ones16 = jnp.ones((16,), jnp.float32)

    for i in range(NC):
        for r in range(3):
            pltpu.async_copy(
                part_hbm.at[pl.ds(i * (3 * NSEG) + r * NSEG + base, _CHUNK)],
                buf.at[pl.ds((i * 3 + r) * _CHUNK, _CHUNK)],
                sem,
            )
    pltpu.make_async_copy(
        part_hbm.at[pl.ds(0, NC * 3 * _CHUNK)], buf, sem
    ).wait()

    for g in range(_CHUNK // 16):
        def red(r):
            a = buf[pl.ds(r * _CHUNK + g * 16, 16)]
            b = buf[pl.ds((3 + r) * _CHUNK + g * 16, 16)]
            return a + b

        s1 = red(0)
        s2 = red(1)
        cnt = jnp.maximum(red(2), ones16)
        osl = pl.ds(g * 16, 16)
        outv0[osl] = s1 / cnt
        outv1[osl] = s2 / cnt

    pltpu.sync_copy(outv0, out_hbm.at[pl.ds(base, _CHUNK)])
    pltpu.sync_copy(outv1, out_hbm.at[pl.ds(NSEG + base, _CHUNK)])


def _to_native_flat(x, ncomp):
    y = x.transpose(1, 0, 2)
    u = y.reshape(ncomp, NT, 128, P)
    z = u.transpose(0, 1, 3, 2)
    return z.reshape(ncomp * NT * P * 128)


def kernel(spex_l1, spex_l2, structure_ids):
    l1_flat = _to_native_flat(spex_l1, C1)
    l2_flat = _to_native_flat(spex_l2, C2)
    ids = structure_ids.astype(jnp.int32)
    part = _partial_kernel(l1_flat, l2_flat, ids)
    out = _finalize_kernel(part.reshape(NC * ROWS * 128))
    means = out.reshape(2, NSEG)[:, :NSTRUCT].T
    return means
